# baseline (speedup 1.0000x reference)
import jax
import jax.numpy as jnp
import numpy as np
from jax import lax
from jax.experimental import pallas as pl
from jax.experimental.pallas import tpu as pltpu
from jax.experimental.pallas import tpu_sc as plsc

B = 16384
F = 26
V = 100000
VPAD = 100096
D = 16

NC = 2
NS = 16
NW = NC * NS
EPW = B // NW
CE = 128
NCHUNK = EPW // CE

CB = 256
RBLK = VPAD // CB
NBLK = F * RBLK
BPW = 320
NRING = 8
TROWS = CB * D // 128
PROWS = F * VPAD // 8
_IOTA_NP = np.arange(16, dtype=np.int32)


def _transpose_body(tab_hbm, out_hbm, in_v, trans_v, sem_i, sem_o):
  c = lax.axis_index("c")
  s = lax.axis_index("s")
  wid = s * NC + c
  base = wid * BPW
  iota = lax.iota(jnp.int32, 16)

  def coords(g):
    ge = lax.rem(g, NBLK)
    f = ge // RBLK
    rb = lax.rem(ge, RBLK)
    return f, rb

  def fire_in(g, slot):
    f, rb = coords(g)
    return pltpu.async_copy(
        tab_hbm.at[f, :, pl.ds(pl.multiple_of(rb * CB, 128), CB)],
        in_v.at[slot], sem_i)

  def fire_out(g, slot):
    f, rb = coords(g)
    orow = pl.multiple_of(f * (VPAD // 8) + rb * TROWS, 8)
    return pltpu.async_copy(trans_v.at[slot],
                            out_hbm.at[pl.ds(orow, TROWS)], sem_o)

  for p in range(NRING):
    fire_in(base + p, p)

  def cycle(cyc, carry):
    g0 = base + cyc * NRING
    for p in range(NRING):
      g = g0 + p
      pltpu.make_async_copy(
          tab_hbm.at[0, :, pl.ds(0, CB)], in_v.at[p], sem_i).wait()
      @pl.when(cyc > 0)
      def _():
        pltpu.make_async_copy(
            trans_v.at[p], out_hbm.at[pl.ds(0, TROWS)], sem_o).wait()
      pslot = jnp.full((16,), p, jnp.int32)
      rhalf = iota >> 3
      colbase = (iota & 7) * 16
      colvs = [colbase + d for d in range(D)]
      for v in range(CB // 16):
        rowv = rhalf + (v * 2)
        for d in range(D):
          val = in_v[p, d, pl.ds(v * 16, 16)]
          plsc.store_scatter(trans_v, [pslot, rowv, colvs[d]], val)
      fire_out(g, p)
      fire_in(g + NRING, p)
    return carry

  lax.fori_loop(0, BPW // NRING, cycle, 0)
  for p in range(NRING):
    pltpu.make_async_copy(
        trans_v.at[p], out_hbm.at[pl.ds(0, TROWS)], sem_o).wait()
    pltpu.make_async_copy(
        tab_hbm.at[0, :, pl.ds(0, CB)], in_v.at[p], sem_i).wait()


def _gather_body(tab_hbm, idx_hbm, out_hbm, idx_v, rows_v, out_v, sem):
  c = lax.axis_index("c")
  s = lax.axis_index("s")
  wid = s * NC + c

  pltpu.sync_copy(idx_hbm.at[wid], idx_v)

  for ch in range(NCHUNK):
    descs = []
    for f in range(F):
      descs.append(pltpu.async_copy(
          tab_hbm.at[idx_v.at[f, pl.ds(ch * CE, CE)]],
          rows_v.at[f],
          sem,
      ))
    for d in descs:
      d.wait()

    def red(e, carry):
      acc = rows_v[0, e, :]
      for f in range(1, F):
        acc = acc + rows_v[f, e, :]
      out_v[e, :] = acc
      return carry

    lax.fori_loop(0, CE, red, 0)
    pltpu.sync_copy(out_v, out_hbm.at[pl.ds(wid * EPW + ch * CE, CE)])


@jax.jit
def _embed_sum(tab3, idx3):
  mesh = plsc.VectorSubcoreMesh(core_axis_name="c", subcore_axis_name="s")
  tabp = pl.kernel(
      _transpose_body,
      out_type=jax.ShapeDtypeStruct((PROWS, 128), jnp.float32),
      mesh=mesh,
      scratch_types=[
          pltpu.VMEM((NRING, 16, CB), jnp.float32),
          pltpu.VMEM((NRING, TROWS, 128), jnp.float32),
          pltpu.SemaphoreType.DMA,
          pltpu.SemaphoreType.DMA,
      ],
      compiler_params=pltpu.CompilerParams(
          needs_layout_passes=False, disable_bounds_checks=True),
  )(tab3)
  tab_flat = tabp.reshape(F * VPAD, D)
  return pl.kernel(
      _gather_body,
      out_type=jax.ShapeDtypeStruct((B, D), jnp.float32),
      mesh=mesh,
      scratch_types=[
          pltpu.VMEM((F, EPW), jnp.int32),
          pltpu.VMEM((F, CE, D), jnp.float32),
          pltpu.VMEM((CE, D), jnp.float32),
          pltpu.SemaphoreType.DMA,
      ],
      compiler_params=pltpu.CompilerParams(use_tc_tiling_on_sc=False),
  )(tab_flat, idx3)


def kernel(x, tables):
  tab3 = jnp.transpose(tables, (0, 2, 1))
  offs = (jnp.arange(F, dtype=jnp.int32) * VPAD)[None, :]
  idx = x + offs
  idx3 = idx.reshape(NW, EPW, F).transpose(0, 2, 1)
  return _embed_sum(tab3, idx3)


# speedup vs baseline: 1.5285x; 1.5285x over previous
import jax
import jax.numpy as jnp
import numpy as np
from jax import lax
from jax.experimental import pallas as pl
from jax.experimental.pallas import tpu as pltpu
from jax.experimental.pallas import tpu_sc as plsc

B = 16384
F = 26
V = 100000
VPAD = 100096
D = 16

NC = 2
NS = 16
NW = NC * NS
EPW = B // NW
CE = 128
NCHUNK = EPW // CE

CB = 256
RBLK = VPAD // CB
NBLK = F * RBLK
BPW = 320
NRING = 4
TROWS = CB * D // 128
PROWS = F * VPAD // 8
_IOTA_NP = np.arange(16, dtype=np.int32)


def _transpose_body(tab_hbm, out_hbm, in_v, trans_v, sem_i, sem_o):
  c = lax.axis_index("c")
  s = lax.axis_index("s")
  wid = s * NC + c
  base = wid * BPW
  iota = lax.iota(jnp.int32, 16)

  def coords(g):
    ge = lax.rem(g, NBLK)
    f = ge // RBLK
    rb = lax.rem(ge, RBLK)
    return f, rb

  def fire_in(g, slot):
    f, rb = coords(g)
    return pltpu.async_copy(
        tab_hbm.at[f, :, pl.ds(pl.multiple_of(rb * CB, 128), CB)],
        in_v.at[slot], sem_i)

  def fire_out(g, slot):
    f, rb = coords(g)
    orow = pl.multiple_of(f * (VPAD // 8) + rb * TROWS, 8)
    return pltpu.async_copy(trans_v.at[slot],
                            out_hbm.at[pl.ds(orow, TROWS)], sem_o)

  for p in range(NRING):
    fire_in(base + p, p)

  def cycle(cyc, carry):
    g0 = base + cyc * NRING
    for p in range(NRING):
      g = g0 + p
      pltpu.make_async_copy(
          tab_hbm.at[0, :, pl.ds(0, CB)], in_v.at[p], sem_i).wait()
      @pl.when(cyc > 0)
      def _():
        pltpu.make_async_copy(
            trans_v.at[p], out_hbm.at[pl.ds(0, TROWS)], sem_o).wait()
      pslot = jnp.full((16,), p, jnp.int32)
      rhalf = iota >> 3
      colbase = (iota & 7) * 16
      colvs = [colbase + d for d in range(D)]
      for v in range(CB // 16):
        rowv = rhalf + (v * 2)
        vals = [in_v[p, d, pl.ds(v * 16, 16)] for d in range(D)]
        for d in range(D):
          plsc.store_scatter(trans_v, [pslot, rowv, colvs[d]], vals[d])
      fire_out(g, p)
      fire_in(g + NRING, p)
    return carry

  lax.fori_loop(0, BPW // NRING, cycle, 0)
  for p in range(NRING):
    pltpu.make_async_copy(
        trans_v.at[p], out_hbm.at[pl.ds(0, TROWS)], sem_o).wait()
    pltpu.make_async_copy(
        tab_hbm.at[0, :, pl.ds(0, CB)], in_v.at[p], sem_i).wait()


def _gather_body(tab_hbm, idx_hbm, out_hbm, idx_v, rows_v, out_v, sem):
  c = lax.axis_index("c")
  s = lax.axis_index("s")
  wid = s * NC + c

  pltpu.sync_copy(idx_hbm.at[wid], idx_v)

  for ch in range(NCHUNK):
    descs = []
    for f in range(F):
      descs.append(pltpu.async_copy(
          tab_hbm.at[idx_v.at[f, pl.ds(ch * CE, CE)]],
          rows_v.at[f],
          sem,
      ))
    for d in descs:
      d.wait()

    def red(e, carry):
      acc = rows_v[0, e, :]
      for f in range(1, F):
        acc = acc + rows_v[f, e, :]
      out_v[e, :] = acc
      return carry

    lax.fori_loop(0, CE, red, 0)
    pltpu.sync_copy(out_v, out_hbm.at[pl.ds(wid * EPW + ch * CE, CE)])


@jax.jit
def _embed_sum(tab3, idx3):
  mesh = plsc.VectorSubcoreMesh(core_axis_name="c", subcore_axis_name="s")
  tabp = pl.kernel(
      _transpose_body,
      out_type=jax.ShapeDtypeStruct((PROWS, 128), jnp.float32),
      mesh=mesh,
      scratch_types=[
          pltpu.VMEM((NRING, 16, CB), jnp.float32),
          pltpu.VMEM((NRING, TROWS, 128), jnp.float32),
          pltpu.SemaphoreType.DMA,
          pltpu.SemaphoreType.DMA,
      ],
      compiler_params=pltpu.CompilerParams(
          needs_layout_passes=False, disable_bounds_checks=True),
  )(tab3)
  tab_flat = tabp.reshape(F * VPAD, D)
  return pl.kernel(
      _gather_body,
      out_type=jax.ShapeDtypeStruct((B, D), jnp.float32),
      mesh=mesh,
      scratch_types=[
          pltpu.VMEM((F, EPW), jnp.int32),
          pltpu.VMEM((F, CE, D), jnp.float32),
          pltpu.VMEM((CE, D), jnp.float32),
          pltpu.SemaphoreType.DMA,
      ],
      compiler_params=pltpu.CompilerParams(use_tc_tiling_on_sc=False),
  )(tab_flat, idx3)


def kernel(x, tables):
  tab3 = jnp.transpose(tables, (0, 2, 1))
  offs = (jnp.arange(F, dtype=jnp.int32) * VPAD)[None, :]
  idx = x + offs
  idx3 = idx.reshape(NW, EPW, F).transpose(0, 2, 1)
  return _embed_sum(tab3, idx3)
